# baseline (speedup 1.0000x reference)
import jax
import jax.numpy as jnp
from jax import lax
from jax.experimental import pallas as pl
from jax.experimental.pallas import tpu as pltpu

B = 16384
L = 200
RBLK = 4096
GRID = B // RBLK


def _tc_kernel(node_ref, items_ref, out_ref):
    node = node_ref[...]
    items = items_ref[...]
    col = lax.broadcasted_iota(jnp.int32, (RBLK, L), 1)
    pick = col == node - 1
    sel = jnp.where(pick, items, jnp.int32(0)).astype(jnp.float32)
    ones = jnp.ones((L, 1), jnp.float32)
    v = jax.lax.dot_general(sel, ones, (((1,), (0,)), ((), ())),
                            precision=jax.lax.Precision.HIGHEST,
                            preferred_element_type=jnp.float32)
    vi = v.astype(jnp.int32)
    out_ref[...] = jnp.where(node != 0, vi, jnp.int32(-1))


@jax.jit
def _decode(node, items):
    out = pl.pallas_call(
        _tc_kernel,
        grid=(GRID,),
        in_specs=[
            pl.BlockSpec((RBLK, 1), lambda i: (i, 0)),
            pl.BlockSpec((RBLK, L), lambda i: (i, 0)),
        ],
        out_specs=pl.BlockSpec((RBLK, 1), lambda i: (i, 0)),
        out_shape=jax.ShapeDtypeStruct((B, 1), jnp.int32),
        compiler_params=pltpu.CompilerParams(
            dimension_semantics=("arbitrary",),
        ),
    )(node, items)
    return jnp.reshape(out, (B,))


def kernel(x_dummy, current_node, item_ids):
    node = current_node.astype(jnp.int32)
    return _decode(node, item_ids.astype(jnp.int32)).astype(item_ids.dtype)


# speedup vs baseline: 3.7886x; 3.7886x over previous
import jax
import jax.numpy as jnp
from jax import lax
from jax.experimental import pallas as pl
from jax.experimental.pallas import tpu as pltpu

B = 16384
L = 200
CBLK = 1024
GRID = B // CBLK
SUB = CBLK // 128


def _tc_kernel(node_ref, items_ref, out_ref):
    items = items_ref[...]
    l_iota = lax.broadcasted_iota(jnp.int32, (L, 128), 0)
    for s in range(SUB):
        node_s = node_ref[s:s + 1, :]
        pick = l_iota == node_s - 1
        sub = items[:, s * 128:(s + 1) * 128]
        sel = jnp.where(pick, sub, jnp.int32(0))
        tot = jnp.sum(sel, axis=0, keepdims=True)
        out_ref[s:s + 1, :] = jnp.where(node_s != 0, tot, jnp.int32(-1))


@jax.jit
def _decode(node2d, items_t):
    return pl.pallas_call(
        _tc_kernel,
        grid=(GRID,),
        in_specs=[
            pl.BlockSpec((SUB, 128), lambda i: (i, 0)),
            pl.BlockSpec((L, CBLK), lambda i: (0, i)),
        ],
        out_specs=pl.BlockSpec((SUB, 128), lambda i: (i, 0)),
        out_shape=jax.ShapeDtypeStruct((B // 128, 128), jnp.int32),
        compiler_params=pltpu.CompilerParams(
            dimension_semantics=("arbitrary",),
        ),
    )(node2d, items_t)


def kernel(x_dummy, current_node, item_ids):
    node2d = jnp.reshape(current_node.astype(jnp.int32), (B // 128, 128))
    items_t = jnp.transpose(item_ids.astype(jnp.int32))
    out = _decode(node2d, items_t)
    return jnp.reshape(out, (B,)).astype(item_ids.dtype)


# speedup vs baseline: 5.5622x; 1.4681x over previous
import jax
import jax.numpy as jnp
from jax import lax
from jax.experimental import pallas as pl
from jax.experimental.pallas import tpu as pltpu

B = 16384
L = 200
CBLK = 2048
GRID = B // CBLK
SUB = CBLK // 128


def _tc_kernel(node_ref, items_ref, out_ref):
    items = items_ref[...]
    l_iota = lax.broadcasted_iota(jnp.int32, (L, 128), 0)
    for s in range(SUB):
        node_s = node_ref[s:s + 1, :]
        pick = l_iota == node_s - 1
        sub = items[:, s * 128:(s + 1) * 128]
        sel = jnp.where(pick, sub, jnp.int32(0))
        tot = jnp.sum(sel, axis=0, keepdims=True)
        out_ref[s:s + 1, :] = jnp.where(node_s != 0, tot, jnp.int32(-1))


@jax.jit
def _decode(node2d, items_t):
    return pl.pallas_call(
        _tc_kernel,
        grid=(GRID,),
        in_specs=[
            pl.BlockSpec((SUB, 128), lambda i: (i, 0)),
            pl.BlockSpec((L, CBLK), lambda i: (0, i)),
        ],
        out_specs=pl.BlockSpec((SUB, 128), lambda i: (i, 0)),
        out_shape=jax.ShapeDtypeStruct((B // 128, 128), jnp.int32),
        compiler_params=pltpu.CompilerParams(
            dimension_semantics=("arbitrary",),
        ),
    )(node2d, items_t)


def kernel(x_dummy, current_node, item_ids):
    node2d = jnp.reshape(current_node.astype(jnp.int32), (B // 128, 128))
    items_t = jnp.transpose(item_ids.astype(jnp.int32))
    out = _decode(node2d, items_t)
    return jnp.reshape(out, (B,)).astype(item_ids.dtype)


# speedup vs baseline: 7.0680x; 1.2707x over previous
import jax
import jax.numpy as jnp
from jax import lax
from jax.experimental import pallas as pl
from jax.experimental.pallas import tpu as pltpu

B = 16384
L = 200
CBLK = 4096
GRID = B // CBLK
SUB = CBLK // 128


def _tc_kernel(node_ref, items_ref, out_ref):
    items = items_ref[...]
    l_iota = lax.broadcasted_iota(jnp.int32, (L, 128), 0)
    for s in range(SUB):
        node_s = node_ref[s:s + 1, :]
        pick = l_iota == node_s - 1
        sub = items[:, s * 128:(s + 1) * 128]
        sel = jnp.where(pick, sub, jnp.int32(0))
        tot = jnp.sum(sel, axis=0, keepdims=True)
        out_ref[s:s + 1, :] = jnp.where(node_s != 0, tot, jnp.int32(-1))


@jax.jit
def _decode(node2d, items_t):
    return pl.pallas_call(
        _tc_kernel,
        grid=(GRID,),
        in_specs=[
            pl.BlockSpec((SUB, 128), lambda i: (i, 0)),
            pl.BlockSpec((L, CBLK), lambda i: (0, i)),
        ],
        out_specs=pl.BlockSpec((SUB, 128), lambda i: (i, 0)),
        out_shape=jax.ShapeDtypeStruct((B // 128, 128), jnp.int32),
        compiler_params=pltpu.CompilerParams(
            dimension_semantics=("arbitrary",),
        ),
    )(node2d, items_t)


def kernel(x_dummy, current_node, item_ids):
    node2d = jnp.reshape(current_node.astype(jnp.int32), (B // 128, 128))
    items_t = jnp.transpose(item_ids.astype(jnp.int32))
    out = _decode(node2d, items_t)
    return jnp.reshape(out, (B,)).astype(item_ids.dtype)


# speedup vs baseline: 7.3517x; 1.0401x over previous
import jax
import jax.numpy as jnp
from jax import lax
from jax.experimental import pallas as pl
from jax.experimental.pallas import tpu as pltpu

B = 16384
L = 200
CBLK = 8192
GRID = B // CBLK
SUB = CBLK // 128


def _tc_kernel(node_ref, items_ref, out_ref):
    items = items_ref[...]
    l_iota = lax.broadcasted_iota(jnp.int32, (L, 128), 0)
    for s in range(SUB):
        node_s = node_ref[s:s + 1, :]
        pick = l_iota == node_s - 1
        sub = items[:, s * 128:(s + 1) * 128]
        sel = jnp.where(pick, sub, jnp.int32(0))
        tot = jnp.sum(sel, axis=0, keepdims=True)
        out_ref[s:s + 1, :] = jnp.where(node_s != 0, tot, jnp.int32(-1))


@jax.jit
def _decode(node2d, items_t):
    return pl.pallas_call(
        _tc_kernel,
        grid=(GRID,),
        in_specs=[
            pl.BlockSpec((SUB, 128), lambda i: (i, 0)),
            pl.BlockSpec((L, CBLK), lambda i: (0, i)),
        ],
        out_specs=pl.BlockSpec((SUB, 128), lambda i: (i, 0)),
        out_shape=jax.ShapeDtypeStruct((B // 128, 128), jnp.int32),
        compiler_params=pltpu.CompilerParams(
            dimension_semantics=("arbitrary",),
        ),
    )(node2d, items_t)


def kernel(x_dummy, current_node, item_ids):
    node2d = jnp.reshape(current_node.astype(jnp.int32), (B // 128, 128))
    items_t = jnp.transpose(item_ids.astype(jnp.int32))
    out = _decode(node2d, items_t)
    return jnp.reshape(out, (B,)).astype(item_ids.dtype)
